# baseline (speedup 1.0000x reference)
import functools

import jax
import jax.numpy as jnp
from jax import lax
from jax.experimental import pallas as pl
from jax.experimental.pallas import tpu as pltpu
from jax.experimental.pallas import tpu_sc as plsc

N = 10000
E = 320000
D = 128
NC = 2
NS = 16
NW = NC * NS
EPW = E // NW
B = 80
PW = 256
NCHUNK = EPW // B
CPS = 5
RPS = 1000


def _proj_body(x_ref, w_ref, b_ref, ptop_ref, pbot_ref):
    p = jnp.dot(x_ref[...], w_ref[...], preferred_element_type=jnp.float32)
    p = p + b_ref[...]
    for t in range(3):
        ptop_ref[t] = p[:, t * D:(t + 1) * D]
        pbot_ref[t] = p[:, 3 * D + t * D:3 * D + (t + 1) * D]


def _node_body(x_ref, zt0_ref, zt1_ref, d0_ref, d1_ref, wt_ref, wb_ref, b_ref,
               out_ref):
    i = pl.program_id(0)
    den = d0_ref[...] + d1_ref[...] + 1e-9
    z = (zt0_ref[...] + zt1_ref[...]) / den
    sel = i < 3
    wt = jnp.where(sel, wt_ref[0], wt_ref[1])
    wb = jnp.where(sel, wb_ref[0], wb_ref[1])
    b = jnp.where(sel, b_ref[0], b_ref[1])
    acc = jnp.dot(x_ref[...], wt, preferred_element_type=jnp.float32)
    acc = acc + jnp.dot(z, wb, preferred_element_type=jnp.float32)
    out_ref[...] = jnp.maximum(acc + b, 0.0)


def _edge_body(cidx_hbm, tblp_hbm, wa_hbm, bav_hbm,
               zt_out, den_out,
               cidx_st, iv_s, iv_d, dv, rs, rd, sbuf, ex, tbuf,
               wa_v, bav_v, z_sh, den_sh, sem):
    cid = lax.axis_index("c")
    sid = lax.axis_index("s")
    wid = sid * NC + cid
    t = jnp.where(wid < 16, 0, jnp.where(wid < 24, 1, 2))
    toff3 = ((3 + t) * N).astype(jnp.int32)

    pltpu.sync_copy(wa_hbm, wa_v)
    pltpu.sync_copy(bav_hbm, bav_v)

    zeros16 = jnp.zeros((16,), jnp.float32)

    def zb_body(r, carry):
        for c in range(8):
            sbuf[r, pl.ds(c * 16, 16)] = zeros16
        return carry

    lax.fori_loop(0, 40, zb_body, 0)
    for i in range(5):
        ex[pl.ds(i * 16, 16)] = zeros16
    for e in range(16):
        tbuf[e, pl.ds(0, 16)] = zeros16

    @pl.when(sid < 10)
    def _():
        z0 = sid * RPS
        for i in range(RPS // 40):
            pltpu.sync_copy(sbuf.at[pl.ds(0, 40), :],
                            z_sh.at[pl.ds(z0 + i * 40, 40), :])
        def dz_body(i2, carry):
            off = pl.multiple_of(z0 + i2 * 40, 8)
            pltpu.sync_copy(ex.at[pl.ds(0, 40)], den_sh.at[pl.ds(off, 40)])
            return carry
        lax.fori_loop(0, RPS // 40, dz_body, 0)

    plsc.subcore_barrier()

    lane = lax.iota(jnp.int32, 16)
    pbase = wid * NCHUNK * PW

    def compute():
        wv = [wa_v[pl.ds(k * 16, 16)] for k in range(8)]
        bav = bav_v[...]

        def edge_partial(row):
            p = None
            for k in range(4):
                sl = pl.ds(k * 16, 16)
                s32 = plsc.bitcast(rs[row, sl], jnp.bfloat16)
                d32 = plsc.bitcast(rd[row, sl], jnp.bfloat16)
                se, so = plsc.unpack(s32, format=plsc.PackFormat.INTERLEAVED)
                de, do = plsc.unpack(d32, format=plsc.PackFormat.INTERLEAVED)
                me = jnp.maximum(se + de, 0.0) * wv[2 * k]
                mo = jnp.maximum(so + do, 0.0) * wv[2 * k + 1]
                m = me + mo
                p = m if p is None else p + m
            return p

        def grp(g, carry):
            g0 = g * 16
            for e in range(16):
                tbuf[e, pl.ds(0, 16)] = edge_partial(g0 + e)
            s = None
            for l in range(16):
                c = plsc.load_gather(tbuf, [lane, jnp.full((16,), l, jnp.int32)])
                s = c if s is None else s + c
            ex[pl.ds(pl.multiple_of(g0, 16), 16)] = jnp.exp(s + bav)
            return carry

        lax.fori_loop(0, B // 16, grp, 0)

        def scale(g, carry):
            exv = ex[pl.ds(pl.multiple_of(g * 16, 16), 16)]
            for e in range(16):
                row = g * 16 + e
                exs = exv[e]
                for k in range(4):
                    xi = plsc.bitcast(rs[row, pl.ds(64 + k * 16, 16)],
                                      jnp.bfloat16)
                    xe, xo = plsc.unpack(xi, format=plsc.PackFormat.INTERLEAVED)
                    sbuf[row, pl.ds(k * 32, 16)] = xe * exs
                    sbuf[row, pl.ds(k * 32 + 16, 16)] = xo * exs
            return carry

        lax.fori_loop(0, B // 16, scale, 0)

    def chunk_body(c, carry):
        koff = pl.multiple_of(lax.rem(c, CPS) * PW, 8)

        @pl.when(lax.rem(c, CPS) == 0)
        def _():
            off = pl.multiple_of(pbase + c * PW, 8)
            pltpu.sync_copy(cidx_hbm.at[pl.ds(off, CPS * PW)], cidx_st)

        for i in range(B // 16):
            sl16 = pl.ds(i * 16, 16)
            iv_s[sl16] = cidx_st[pl.ds(koff + i * 16, 16)]
            dval = cidx_st[pl.ds(koff + B + i * 16, 16)]
            iv_d[sl16] = dval
            dv[sl16] = dval - toff3

        cp_s = pltpu.async_copy(tblp_hbm.at[iv_s], rs, sem)
        cp_d = pltpu.async_copy(tblp_hbm.at[iv_d], rd, sem)
        cp_s.wait()
        cp_d.wait()

        compute()

        cs_z = pltpu.async_copy(sbuf, z_sh.at[dv], sem, add=True)
        cs_d = pltpu.async_copy(ex, den_sh.at[dv], sem, add=True)
        cs_z.wait()
        cs_d.wait()
        return carry

    lax.fori_loop(0, NCHUNK, chunk_body, 0)

    plsc.subcore_barrier()

    @pl.when(sid < 10)
    def _():
        r0 = sid * RPS
        pltpu.sync_copy(z_sh.at[pl.ds(r0, RPS), :],
                        zt_out.at[cid, pl.ds(r0, RPS), :])

    @pl.when(sid == 0)
    def _():
        pltpu.sync_copy(den_sh, den_out.at[cid])


_edge_call = functools.partial(
    pl.kernel,
    out_type=[
        jax.ShapeDtypeStruct((NC, N, D), jnp.float32),
        jax.ShapeDtypeStruct((NC, N), jnp.float32),
    ],
    mesh=plsc.VectorSubcoreMesh(core_axis_name="c", subcore_axis_name="s"),
    compiler_params=pltpu.CompilerParams(needs_layout_passes=False),
    scratch_types=[
        pltpu.VMEM((CPS * PW,), jnp.int32),
        pltpu.VMEM((B,), jnp.int32),
        pltpu.VMEM((B,), jnp.int32),
        pltpu.VMEM((B,), jnp.int32),
        pltpu.VMEM((B, D), jnp.int32),
        pltpu.VMEM((B, D), jnp.int32),
        pltpu.VMEM((B, D), jnp.float32),
        pltpu.VMEM((B,), jnp.float32),
        pltpu.VMEM((16, 17), jnp.float32),
        pltpu.VMEM((D,), jnp.float32),
        pltpu.VMEM((16,), jnp.float32),
        pltpu.VMEM_SHARED((N, D), jnp.float32),
        pltpu.VMEM_SHARED((N,), jnp.float32),
        pltpu.SemaphoreType.DMA,
    ],
)(_edge_body)


def kernel(x, edge_index, W_hh, b_hh, W_oo, b_oo, W_ho, b_ho, W_a, b_a,
           W_hn, b_hn, W_on, b_on):
    R = 1000

    wfull = jnp.concatenate(
        [W_hh[:D], W_oo[:D], W_ho[:D], W_hh[D:], W_oo[D:], W_ho[D:]], axis=1)
    bfull = jnp.concatenate(
        [b_hh, b_oo, b_ho, jnp.zeros((3 * D,), jnp.float32)]).reshape(1, 6 * D)

    ptop, pbot = pl.pallas_call(
        _proj_body,
        grid=(N // R,),
        in_specs=[
            pl.BlockSpec((R, D), lambda i: (i, 0)),
            pl.BlockSpec((D, 6 * D), lambda i: (0, 0)),
            pl.BlockSpec((1, 6 * D), lambda i: (0, 0)),
        ],
        out_specs=[
            pl.BlockSpec((3, R, D), lambda i: (0, i, 0)),
            pl.BlockSpec((3, R, D), lambda i: (0, i, 0)),
        ],
        out_shape=[
            jax.ShapeDtypeStruct((3, N, D), jnp.float32),
            jax.ShapeDtypeStruct((3, N, D), jnp.float32),
        ],
    )(x, wfull, bfull)

    src = edge_index[0]
    dst = edge_index[1]
    toff = jnp.concatenate([
        jnp.zeros((E // 2,), jnp.int32),
        jnp.full((E // 4,), N, jnp.int32),
        jnp.full((E - E // 2 - E // 4,), 2 * N, jnp.int32),
    ])
    sadj = (src + toff).reshape(NW, NCHUNK, B)
    dadj = (dst + toff + 3 * N).reshape(NW, NCHUNK, B)
    xidx = src.reshape(NW, NCHUNK, B)
    pad = jnp.zeros((NW, NCHUNK, PW - 3 * B), jnp.int32)
    cidx = jnp.concatenate([sadj, dadj, xidx, pad], axis=2).reshape(-1)
    x_bf = jnp.broadcast_to(x.astype(jnp.bfloat16), (3, N, D))
    ta = jnp.concatenate([ptop.astype(jnp.bfloat16), x_bf], axis=2)
    tb = jnp.concatenate([pbot.astype(jnp.bfloat16), x_bf], axis=2)
    tblp = lax.bitcast_convert_type(
        jnp.concatenate([ta, tb]).reshape(6 * N, D, 2), jnp.int32)
    wa4 = W_a[:, 0].reshape(4, 16, 2)
    wa = jnp.concatenate([wa4[:, :, 0], wa4[:, :, 1]], axis=1).reshape(D)
    bav = jnp.full((16,), b_a[0], jnp.float32)

    zt, den = _edge_call(cidx, tblp, wa, bav)

    perm4 = jnp.arange(D).reshape(4, 16, 2)
    perm = jnp.concatenate([perm4[:, :, 0], perm4[:, :, 1]], axis=1).reshape(D)
    wt_s = jnp.stack([W_hn[:D], W_on[:D]])
    wb_s = jnp.stack([W_hn[D:][perm], W_on[D:][perm]])
    b_s = jnp.stack([b_hn.reshape(1, D), b_on.reshape(1, D)])

    out = pl.pallas_call(
        _node_body,
        grid=(N // R,),
        in_specs=[
            pl.BlockSpec((R, D), lambda i: (i, 0)),
            pl.BlockSpec((R, D), lambda i: (i, 0)),
            pl.BlockSpec((R, D), lambda i: (i, 0)),
            pl.BlockSpec((R, 1), lambda i: (i, 0)),
            pl.BlockSpec((R, 1), lambda i: (i, 0)),
            pl.BlockSpec((2, D, D), lambda i: (0, 0, 0)),
            pl.BlockSpec((2, D, D), lambda i: (0, 0, 0)),
            pl.BlockSpec((2, 1, D), lambda i: (0, 0, 0)),
        ],
        out_specs=pl.BlockSpec((R, D), lambda i: (i, 0)),
        out_shape=jax.ShapeDtypeStruct((N, D), jnp.float32),
    )(x, zt[0], zt[1], den[0].reshape(N, 1), den[1].reshape(N, 1),
      wt_s, wb_s, b_s)
    return out


# speedup vs baseline: 1.6332x; 1.6332x over previous
import functools

import jax
import jax.numpy as jnp
from jax import lax
from jax.experimental import pallas as pl
from jax.experimental.pallas import tpu as pltpu
from jax.experimental.pallas import tpu_sc as plsc

N = 10000
E = 320000
D = 128
NC = 2
NS = 16
NW = NC * NS
EPW = E // NW
B = 80
PW = 256
NCHUNK = EPW // B
CPS = 25
RPS = 1000


def _proj_body(x_ref, w_ref, b_ref, ptop_ref, pbot_ref):
    p = jnp.dot(x_ref[...], w_ref[...], preferred_element_type=jnp.float32)
    p = p + b_ref[...]
    for t in range(3):
        ptop_ref[t] = p[:, t * D:(t + 1) * D]
        pbot_ref[t] = p[:, 3 * D + t * D:3 * D + (t + 1) * D]


def _node_body(x_ref, zt0_ref, zt1_ref, d0_ref, d1_ref, wt_ref, wb_ref, b_ref,
               out_ref):
    i = pl.program_id(0)
    den = d0_ref[...] + d1_ref[...] + 1e-9
    z = (zt0_ref[...] + zt1_ref[...]) / den
    sel = i < 3
    wt = jnp.where(sel, wt_ref[0], wt_ref[1])
    wb = jnp.where(sel, wb_ref[0], wb_ref[1])
    b = jnp.where(sel, b_ref[0], b_ref[1])
    acc = jnp.dot(x_ref[...], wt, preferred_element_type=jnp.float32)
    acc = acc + jnp.dot(z, wb, preferred_element_type=jnp.float32)
    out_ref[...] = jnp.maximum(acc + b, 0.0)


def _edge_body(cidx_hbm, tbl_hbm, wa_hbm, bav_hbm,
               zt_out, den_out,
               cidx_st, iv_s, iv_d, iv_x, dv, rs, rd, rx, ex, tbuf,
               wa_v, bav_v, z_sh, den_sh, sem):
    cid = lax.axis_index("c")
    sid = lax.axis_index("s")
    wid = sid * NC + cid
    t = jnp.where(wid < 16, 0, jnp.where(wid < 24, 1, 2))
    toff3 = ((3 + t) * N).astype(jnp.int32)

    pltpu.sync_copy(wa_hbm, wa_v)
    pltpu.sync_copy(bav_hbm, bav_v)

    zeros16 = jnp.zeros((16,), jnp.float32)

    def zb_body(r, carry):
        for c in range(8):
            rx[r, pl.ds(c * 16, 16)] = zeros16
        return carry

    lax.fori_loop(0, 40, zb_body, 0)
    for i in range(5):
        ex[pl.ds(i * 16, 16)] = zeros16
    for e in range(16):
        tbuf[e, pl.ds(0, 16)] = zeros16

    @pl.when(sid < 10)
    def _():
        z0 = sid * RPS
        for i in range(RPS // 40):
            pltpu.sync_copy(rx.at[pl.ds(0, 40), :],
                            z_sh.at[pl.ds(z0 + i * 40, 40), :])
        def dz_body(i2, carry):
            off = pl.multiple_of(z0 + i2 * 40, 8)
            pltpu.sync_copy(ex.at[pl.ds(0, 40)], den_sh.at[pl.ds(off, 40)])
            return carry
        lax.fori_loop(0, RPS // 40, dz_body, 0)

    plsc.subcore_barrier()

    lane = lax.iota(jnp.int32, 16)
    pbase = wid * NCHUNK * PW

    def compute():
        wv = [wa_v[pl.ds(k * 16, 16)] for k in range(8)]
        bav = bav_v[...]

        def edge_partial(row):
            p = None
            for k in range(8):
                sl = pl.ds(k * 16, 16)
                u = jnp.maximum(rs[row, sl] + rd[row, sl], 0.0)
                m = u * wv[k]
                p = m if p is None else p + m
            return p

        def grp(g, carry):
            g0 = g * 16
            for e in range(16):
                tbuf[e, pl.ds(0, 16)] = edge_partial(g0 + e)
            s = None
            for l in range(16):
                c = plsc.load_gather(tbuf, [lane, jnp.full((16,), l, jnp.int32)])
                s = c if s is None else s + c
            ex[pl.ds(pl.multiple_of(g0, 16), 16)] = jnp.exp(s + bav)
            return carry

        lax.fori_loop(0, B // 16, grp, 0)

        def scale(g, carry):
            exv = ex[pl.ds(pl.multiple_of(g * 16, 16), 16)]
            for e in range(16):
                row = g * 16 + e
                exs = exv[e]
                for k in range(8):
                    sl = pl.ds(k * 16, 16)
                    rx[row, sl] = rx[row, sl] * exs
            return carry

        lax.fori_loop(0, B // 16, scale, 0)

    def chunk_body(c, carry):
        koff = pl.multiple_of(lax.rem(c, CPS) * PW, 8)

        @pl.when(lax.rem(c, CPS) == 0)
        def _():
            off = pl.multiple_of(pbase + c * PW, 8)
            pltpu.sync_copy(cidx_hbm.at[pl.ds(off, CPS * PW)], cidx_st)

        for i in range(B // 16):
            sl16 = pl.ds(i * 16, 16)
            iv_s[sl16] = cidx_st[pl.ds(koff + i * 16, 16)]
            dval = cidx_st[pl.ds(koff + B + i * 16, 16)]
            iv_d[sl16] = dval
            dv[sl16] = dval - toff3
            iv_x[sl16] = cidx_st[pl.ds(koff + 2 * B + i * 16, 16)]

        cp_s = pltpu.async_copy(tbl_hbm.at[iv_s], rs, sem)
        cp_d = pltpu.async_copy(tbl_hbm.at[iv_d], rd, sem)
        cp_x = pltpu.async_copy(tbl_hbm.at[iv_x], rx, sem)
        cp_s.wait()
        cp_d.wait()
        cp_x.wait()

        compute()

        cs_z = pltpu.async_copy(rx, z_sh.at[dv], sem, add=True)
        cs_d = pltpu.async_copy(ex, den_sh.at[dv], sem, add=True)
        cs_z.wait()
        cs_d.wait()
        return carry

    lax.fori_loop(0, NCHUNK, chunk_body, 0)

    plsc.subcore_barrier()

    @pl.when(sid < 10)
    def _():
        r0 = sid * RPS
        pltpu.sync_copy(z_sh.at[pl.ds(r0, RPS), :],
                        zt_out.at[cid, pl.ds(r0, RPS), :])

    @pl.when(sid == 0)
    def _():
        pltpu.sync_copy(den_sh, den_out.at[cid])


_edge_call = functools.partial(
    pl.kernel,
    out_type=[
        jax.ShapeDtypeStruct((NC, N, D), jnp.float32),
        jax.ShapeDtypeStruct((NC, N), jnp.float32),
    ],
    mesh=plsc.VectorSubcoreMesh(core_axis_name="c", subcore_axis_name="s"),
    compiler_params=pltpu.CompilerParams(needs_layout_passes=False),
    scratch_types=[
        pltpu.VMEM((CPS * PW,), jnp.int32),
        pltpu.VMEM((B,), jnp.int32),
        pltpu.VMEM((B,), jnp.int32),
        pltpu.VMEM((B,), jnp.int32),
        pltpu.VMEM((B,), jnp.int32),
        pltpu.VMEM((B, D), jnp.float32),
        pltpu.VMEM((B, D), jnp.float32),
        pltpu.VMEM((B, D), jnp.float32),
        pltpu.VMEM((B,), jnp.float32),
        pltpu.VMEM((16, 17), jnp.float32),
        pltpu.VMEM((D,), jnp.float32),
        pltpu.VMEM((16,), jnp.float32),
        pltpu.VMEM_SHARED((N, D), jnp.float32),
        pltpu.VMEM_SHARED((N,), jnp.float32),
        pltpu.SemaphoreType.DMA,
    ],
)(_edge_body)


def kernel(x, edge_index, W_hh, b_hh, W_oo, b_oo, W_ho, b_ho, W_a, b_a,
           W_hn, b_hn, W_on, b_on):
    R = 1000

    wfull = jnp.concatenate(
        [W_hh[:D], W_oo[:D], W_ho[:D], W_hh[D:], W_oo[D:], W_ho[D:]], axis=1)
    bfull = jnp.concatenate(
        [b_hh, b_oo, b_ho, jnp.zeros((3 * D,), jnp.float32)]).reshape(1, 6 * D)

    ptop, pbot = pl.pallas_call(
        _proj_body,
        grid=(N // R,),
        in_specs=[
            pl.BlockSpec((R, D), lambda i: (i, 0)),
            pl.BlockSpec((D, 6 * D), lambda i: (0, 0)),
            pl.BlockSpec((1, 6 * D), lambda i: (0, 0)),
        ],
        out_specs=[
            pl.BlockSpec((3, R, D), lambda i: (0, i, 0)),
            pl.BlockSpec((3, R, D), lambda i: (0, i, 0)),
        ],
        out_shape=[
            jax.ShapeDtypeStruct((3, N, D), jnp.float32),
            jax.ShapeDtypeStruct((3, N, D), jnp.float32),
        ],
    )(x, wfull, bfull)

    src = edge_index[0]
    dst = edge_index[1]
    toff = jnp.concatenate([
        jnp.zeros((E // 2,), jnp.int32),
        jnp.full((E // 4,), N, jnp.int32),
        jnp.full((E - E // 2 - E // 4,), 2 * N, jnp.int32),
    ])
    sadj = (src + toff).reshape(NW, NCHUNK, B)
    dadj = (dst + toff + 3 * N).reshape(NW, NCHUNK, B)
    xidx = (src + 6 * N).reshape(NW, NCHUNK, B)
    pad = jnp.zeros((NW, NCHUNK, PW - 3 * B), jnp.int32)
    cidx = jnp.concatenate([sadj, dadj, xidx, pad], axis=2).reshape(-1)
    tbl = jnp.concatenate([ptop.reshape(3 * N, D), pbot.reshape(3 * N, D), x])
    wa = W_a[:, 0]
    bav = jnp.full((16,), b_a[0], jnp.float32)

    zt, den = _edge_call(cidx, tbl, wa, bav)

    wt_s = jnp.stack([W_hn[:D], W_on[:D]])
    wb_s = jnp.stack([W_hn[D:], W_on[D:]])
    b_s = jnp.stack([b_hn.reshape(1, D), b_on.reshape(1, D)])

    out = pl.pallas_call(
        _node_body,
        grid=(N // R,),
        in_specs=[
            pl.BlockSpec((R, D), lambda i: (i, 0)),
            pl.BlockSpec((R, D), lambda i: (i, 0)),
            pl.BlockSpec((R, D), lambda i: (i, 0)),
            pl.BlockSpec((R, 1), lambda i: (i, 0)),
            pl.BlockSpec((R, 1), lambda i: (i, 0)),
            pl.BlockSpec((2, D, D), lambda i: (0, 0, 0)),
            pl.BlockSpec((2, D, D), lambda i: (0, 0, 0)),
            pl.BlockSpec((2, 1, D), lambda i: (0, 0, 0)),
        ],
        out_specs=pl.BlockSpec((R, D), lambda i: (i, 0)),
        out_shape=jax.ShapeDtypeStruct((N, D), jnp.float32),
    )(x, zt[0], zt[1], den[0].reshape(N, 1), den[1].reshape(N, 1),
      wt_s, wb_s, b_s)
    return out
